# baseline (speedup 1.0000x reference)
import jax
import jax.numpy as jnp
from jax.experimental import pallas as pl
from jax.experimental.pallas import tpu as pltpu

B, H, S, D, Q = 1, 8, 2048, 128, 16
W = 256


def _body(*refs):
    pos_ref = refs[0]
    ins = refs[1:17]
    outs = refs[17:25]
    v_win = refs[25:29]
    v_slc = refs[29:33]
    sems = refs[33:]
    bulk_sem = sems[0:8]
    k_sem = sems[8:12]
    win_sem = sems[12:16]
    slc_sem = sems[16:20]
    vout_sem = sems[20:24]

    pos = pos_ref[0]
    cw = jnp.minimum((pos // 128) * 128, S - W)
    off = pos - cw

    bulk = []
    for j in range(8):
        c = pltpu.make_async_copy(ins[2 * j], outs[j], bulk_sem[j])
        c.start()
        bulk.append(c)

    win_cp, slc_cp = [], []
    for l in range(4):
        cv_in = ins[4 * l + 2]
        sv_in = ins[4 * l + 3]
        c = pltpu.make_async_copy(cv_in.at[0, :, :, pl.ds(cw, W)], v_win[l], win_sem[l])
        c.start()
        win_cp.append(c)
        c = pltpu.make_async_copy(sv_in.at[0], v_slc[l], slc_sem[l])
        c.start()
        slc_cp.append(c)

    k_cp = []
    for l in range(4):
        bulk[2 * l].wait()
        c = pltpu.make_async_copy(
            ins[4 * l + 1], outs[2 * l].at[:, :, pl.ds(pos, Q), :], k_sem[l]
        )
        c.start()
        k_cp.append(c)

    lane = jax.lax.broadcasted_iota(jnp.int32, (H, D, W), 2)
    mask = (lane >= off) & (lane < off + Q)
    vout_cp = []
    for l in range(4):
        win_cp[l].wait()
        slc_cp[l].wait()
        padded = jnp.pad(v_slc[l][...], ((0, 0), (0, 0), (0, W - Q)))
        rolled = pltpu.roll(padded, off, 2)
        v_win[l][...] = jnp.where(mask, rolled, v_win[l][...])
        bulk[2 * l + 1].wait()
        c = pltpu.make_async_copy(
            v_win[l], outs[2 * l + 1].at[0, :, :, pl.ds(cw, W)], vout_sem[l]
        )
        c.start()
        vout_cp.append(c)

    for c in k_cp:
        c.wait()
    for c in vout_cp:
        c.wait()


def kernel(input_pos, kv_cache_k_0, kv_slice_k_0, kv_cache_v_0, kv_slice_v_0, kv_cache_k_1, kv_slice_k_1, kv_cache_v_1, kv_slice_v_1, kv_cache_k_2, kv_slice_k_2, kv_cache_v_2, kv_slice_v_2, kv_cache_k_3, kv_slice_k_3, kv_cache_v_3, kv_slice_v_3):
    caches_and_slices = (
        kv_cache_k_0, kv_slice_k_0, kv_cache_v_0, kv_slice_v_0,
        kv_cache_k_1, kv_slice_k_1, kv_cache_v_1, kv_slice_v_1,
        kv_cache_k_2, kv_slice_k_2, kv_cache_v_2, kv_slice_v_2,
        kv_cache_k_3, kv_slice_k_3, kv_cache_v_3, kv_slice_v_3,
    )
    k_shape = jax.ShapeDtypeStruct((B, H, S, D), jnp.float32)
    v_shape = jax.ShapeDtypeStruct((B, H, D, S), jnp.float32)
    out_shape = (k_shape, v_shape) * 4

    outs = pl.pallas_call(
        _body,
        in_specs=[pl.BlockSpec(memory_space=pltpu.SMEM)]
        + [pl.BlockSpec(memory_space=pl.ANY)] * 16,
        out_specs=tuple(pl.BlockSpec(memory_space=pl.ANY) for _ in range(8)),
        out_shape=out_shape,
        scratch_shapes=[pltpu.VMEM((H, D, W), jnp.float32) for _ in range(4)]
        + [pltpu.VMEM((H, D, Q), jnp.float32) for _ in range(4)]
        + [pltpu.SemaphoreType.DMA] * 24,
    )(input_pos.astype(jnp.int32), *caches_and_slices)
    return tuple(outs)


# speedup vs baseline: 38.5308x; 38.5308x over previous
import jax
import jax.numpy as jnp
from jax.experimental import pallas as pl
from jax.experimental.pallas import tpu as pltpu

B, H, S, D, Q = 1, 8, 2048, 128, 16
C = 256
G = S // C


def _body(pos_ref, *refs):
    ins = refs[0:16]
    outs = refs[16:24]
    pos = pos_ref[0]
    i = pl.program_id(0)
    base = i * C

    for l in range(4):
        ck, sk, cv, sv = ins[4 * l], ins[4 * l + 1], ins[4 * l + 2], ins[4 * l + 3]
        ko, vo = outs[2 * l], outs[2 * l + 1]

        ko[...] = ck[...]
        r0 = pos - base
        for q in range(Q):
            rq = r0 + q

            @pl.when((rq >= 0) & (rq < C))
            def _(l=l, q=q, rq=rq, ko=ko, sk=sk):
                ko[0, :, pl.ds(jnp.clip(rq, 0, C - 1), 1), :] = sk[0, :, pl.ds(q, 1), :]

        shift = jnp.mod(pos - base, C)
        padded = jnp.pad(sv[0][...], ((0, 0), (0, 0), (0, C - Q)))
        rolled = pltpu.roll(padded, shift, 2)
        lane_g = jax.lax.broadcasted_iota(jnp.int32, (H, D, C), 2) + base
        mask = (lane_g >= pos) & (lane_g < pos + Q)
        vo[...] = jnp.where(mask[None], rolled[None], cv[...])


def kernel(input_pos, kv_cache_k_0, kv_slice_k_0, kv_cache_v_0, kv_slice_v_0, kv_cache_k_1, kv_slice_k_1, kv_cache_v_1, kv_slice_v_1, kv_cache_k_2, kv_slice_k_2, kv_cache_v_2, kv_slice_v_2, kv_cache_k_3, kv_slice_k_3, kv_cache_v_3, kv_slice_v_3):
    caches_and_slices = (
        kv_cache_k_0, kv_slice_k_0, kv_cache_v_0, kv_slice_v_0,
        kv_cache_k_1, kv_slice_k_1, kv_cache_v_1, kv_slice_v_1,
        kv_cache_k_2, kv_slice_k_2, kv_cache_v_2, kv_slice_v_2,
        kv_cache_k_3, kv_slice_k_3, kv_cache_v_3, kv_slice_v_3,
    )
    k_shape = jax.ShapeDtypeStruct((B, H, S, D), jnp.float32)
    v_shape = jax.ShapeDtypeStruct((B, H, D, S), jnp.float32)
    out_shape = (k_shape, v_shape) * 4

    k_cache_spec = pl.BlockSpec((B, H, C, D), lambda i, p: (0, 0, i, 0))
    k_slice_spec = pl.BlockSpec((B, H, Q, D), lambda i, p: (0, 0, 0, 0))
    v_cache_spec = pl.BlockSpec((B, H, D, C), lambda i, p: (0, 0, 0, i))
    v_slice_spec = pl.BlockSpec((B, H, D, Q), lambda i, p: (0, 0, 0, 0))

    grid_spec = pltpu.PrefetchScalarGridSpec(
        num_scalar_prefetch=1,
        grid=(G,),
        in_specs=[k_cache_spec, k_slice_spec, v_cache_spec, v_slice_spec] * 4,
        out_specs=[k_cache_spec, v_cache_spec] * 4,
    )

    outs = pl.pallas_call(
        _body,
        grid_spec=grid_spec,
        out_shape=out_shape,
        compiler_params=pltpu.CompilerParams(
            dimension_semantics=("arbitrary",),
        ),
    )(input_pos.astype(jnp.int32), *caches_and_slices)
    return tuple(outs)
